# Initial kernel scaffold; baseline (speedup 1.0000x reference)
import jax
import jax.numpy as jnp
from jax.experimental import pallas as pl


def kernel(input_node, inputad, res, resmask, W1, att_src1, att_dst1, b1, W2, att_src2, att_dst2, b2, lin_w, lin_b):
    raise NotImplementedError("write your pallas kernel here")


# speedup vs baseline: 1.3285x; 1.3285x over previous
import functools

import jax
import jax.numpy as jnp
from jax.experimental import pallas as pl
from jax.experimental.pallas import tpu as pltpu

N = 10000
EMB = 128
NEG_SLOPE = 0.2
NPAD = 10016


def _pre_body(x_ref, w_ref, asv_ref, adv_ref, h_ref, as_ref, ad_ref, ub_ref):
    h = jnp.dot(x_ref[:], w_ref[:], preferred_element_type=jnp.float32)
    h_ref[:] = h
    a_s = jnp.dot(h, asv_ref[:], preferred_element_type=jnp.float32)
    a_d = jnp.dot(h, adv_ref[:], preferred_element_type=jnp.float32)
    pad = jnp.zeros((NPAD - N, 1), jnp.float32)
    as_ref[:] = jnp.concatenate([a_s, pad], axis=0)
    ad_ref[:] = jnp.concatenate([a_d, pad], axis=0)
    ub = jnp.max(a_s) + jnp.max(a_d)
    ub = jnp.where(ub < 0, NEG_SLOPE * ub, ub)
    ub_ref[:] = jnp.full((1, 16), ub, jnp.float32)


def _tc_pre(x, W, att_src, att_dst):
    out = pl.pallas_call(
        _pre_body,
        out_shape=(
            jax.ShapeDtypeStruct((x.shape[0], EMB), jnp.float32),
            jax.ShapeDtypeStruct((NPAD, 1), jnp.float32),
            jax.ShapeDtypeStruct((NPAD, 1), jnp.float32),
            jax.ShapeDtypeStruct((1, 16), jnp.float32),
        ),
    )(x, W, att_src.reshape(EMB, 1), att_dst.reshape(EMB, 1))
    h, a_s, a_d, ub = out
    return h, a_s[:, 0], a_d[:, 0], ub[0]


def _mid_body(p0_ref, p1_ref, b_ref, w_ref, asv_ref, adv_ref,
              h_ref, as_ref, ad_ref, ub_ref):
    x = jax.nn.relu(p0_ref[:] + p1_ref[:] + b_ref[:])
    h = jnp.dot(x, w_ref[:], preferred_element_type=jnp.float32)
    h_ref[:] = h
    a_s = jnp.dot(h, asv_ref[:], preferred_element_type=jnp.float32)
    a_d = jnp.dot(h, adv_ref[:], preferred_element_type=jnp.float32)
    pad = jnp.zeros((NPAD - N, 1), jnp.float32)
    as_ref[:] = jnp.concatenate([a_s, pad], axis=0)
    ad_ref[:] = jnp.concatenate([a_d, pad], axis=0)
    ub = jnp.max(a_s) + jnp.max(a_d)
    ub = jnp.where(ub < 0, NEG_SLOPE * ub, ub)
    ub_ref[:] = jnp.full((1, 16), ub, jnp.float32)


def _tc_mid(p0, p1, b, W, att_src, att_dst):
    out = pl.pallas_call(
        _mid_body,
        out_shape=(
            jax.ShapeDtypeStruct((N, EMB), jnp.float32),
            jax.ShapeDtypeStruct((NPAD, 1), jnp.float32),
            jax.ShapeDtypeStruct((NPAD, 1), jnp.float32),
            jax.ShapeDtypeStruct((1, 16), jnp.float32),
        ),
    )(p0, p1, b.reshape(1, EMB), W,
      att_src.reshape(EMB, 1), att_dst.reshape(EMB, 1))
    h, a_s, a_d, ub = out
    return h, a_s[:, 0], a_d[:, 0], ub[0]


def _fin_body(p0_ref, p1_ref, b_ref, lw_ref, lb_ref, res_ref, rm_ref,
              x2_ref, sm_ref, loss_ref):
    x2 = jax.nn.relu(p0_ref[:] + p1_ref[:] + b_ref[:])
    x2_ref[:] = x2
    li = jnp.dot(x2, lw_ref[:], preferred_element_type=jnp.float32) + lb_ref[0, 0]
    li = jnp.where(rm_ref[:] == 0, -1e9, li)
    m = jnp.max(li)
    ex = jnp.exp(li - m)
    sm = ex / jnp.sum(ex)
    sm_ref[:] = sm
    loss = -jnp.sum(jnp.log(jnp.clip(sm, 1e-10, 1.0)) * res_ref[:])
    loss_ref[:] = jnp.full((1, 1), loss, jnp.float32)


def _tc_fin(p0, p1, b, lin_w, lin_b, res, resmask):
    x2, sm, loss = pl.pallas_call(
        _fin_body,
        out_shape=(
            jax.ShapeDtypeStruct((N, EMB), jnp.float32),
            jax.ShapeDtypeStruct((N, 1), jnp.float32),
            jax.ShapeDtypeStruct((1, 1), jnp.float32),
        ),
    )(p0, p1, b.reshape(1, EMB), lin_w, lin_b.reshape(1, 1),
      res.reshape(N, 1), resmask.reshape(N, 1))
    return x2, sm[:, 0], loss.reshape(())


def _edge_phase(h, src, dst, a_s, a_d, ub):
    e = a_s[src] + a_d[dst]
    e = jnp.where(e < 0, NEG_SLOPE * e, e)
    ex = jnp.exp(e - ub[0])
    denom = jax.ops.segment_sum(ex, dst, num_segments=N)
    alpha = ex / denom[dst]
    out = jax.ops.segment_sum(h[src] * alpha[:, None], dst, num_segments=N)
    return out


def kernel(input_node, inputad, res, resmask, W1, att_src1, att_dst1, b1,
           W2, att_src2, att_dst2, b2, lin_w, lin_b):
    loop = jnp.arange(N, dtype=jnp.int32)
    src = jnp.concatenate([inputad[0], loop])
    dst = jnp.concatenate([inputad[1], loop])

    h1, as1, ad1, ub1 = _tc_pre(input_node, W1, att_src1, att_dst1)
    p1a = _edge_phase(h1, src, dst, as1, ad1, ub1)
    z = jnp.zeros_like(p1a)
    h2, as2, ad2, ub2 = _tc_mid(p1a, z, b1, W2, att_src2, att_dst2)
    p2a = _edge_phase(h2, src, dst, as2, ad2, ub2)
    x2, sm, loss = _tc_fin(p2a, z, b2, lin_w, lin_b, res, resmask)
    return (loss, sm, x2, resmask, x2)


# speedup vs baseline: 30.1063x; 22.6614x over previous
import functools

import jax
import jax.numpy as jnp
from jax import lax
from jax.experimental import pallas as pl
from jax.experimental.pallas import tpu as pltpu
from jax.experimental.pallas import tpu_sc as plsc

N = 10000
EMB = 128
NEG_SLOPE = 0.2
NPAD = 10240
EP = 331776
EROWS = EP // 128
TROWS = EROWS // 32
SL = NPAD // 16


def _proj(h, asv_ref, adv_ref):
    a_s = jnp.dot(h, asv_ref[:], preferred_element_type=jnp.float32)
    a_d = jnp.dot(h, adv_ref[:], preferred_element_type=jnp.float32)
    pad = jnp.zeros((NPAD - N, 1), jnp.float32)
    ub = jnp.max(a_s) + jnp.max(a_d)
    ub = jnp.where(ub < 0, NEG_SLOPE * ub, ub)
    return (jnp.concatenate([a_s, pad], axis=0),
            jnp.concatenate([a_d, pad], axis=0),
            jnp.full((1, 16), ub, jnp.float32))


def _pre_body(x_ref, w_ref, asv_ref, adv_ref, h_ref, as_ref, ad_ref, ub_ref):
    h = jnp.dot(x_ref[:], w_ref[:], preferred_element_type=jnp.float32)
    h_ref[:] = h
    as_ref[:], ad_ref[:], ub_ref[:] = _proj(h, asv_ref, adv_ref)


def _tc_pre(x, W, att_src, att_dst):
    h, a_s, a_d, ub = pl.pallas_call(
        _pre_body,
        out_shape=(
            jax.ShapeDtypeStruct((N, EMB), jnp.float32),
            jax.ShapeDtypeStruct((NPAD, 1), jnp.float32),
            jax.ShapeDtypeStruct((NPAD, 1), jnp.float32),
            jax.ShapeDtypeStruct((1, 16), jnp.float32),
        ),
    )(x, W, att_src.reshape(EMB, 1), att_dst.reshape(EMB, 1))
    return h, a_s[:, 0], a_d[:, 0], ub.reshape(16)


def _mid_body(n0_ref, n1_ref, d0_ref, d1_ref, b_ref, w_ref, asv_ref, adv_ref,
              h_ref, as_ref, ad_ref, ub_ref):
    inv = 1.0 / (d0_ref[:] + d1_ref[:])
    x = jax.nn.relu((n0_ref[:] + n1_ref[:]) * inv + b_ref[:])
    h = jnp.dot(x, w_ref[:], preferred_element_type=jnp.float32)
    h_ref[:] = h
    as_ref[:], ad_ref[:], ub_ref[:] = _proj(h, asv_ref, adv_ref)


def _tc_mid(n0, n1, d0, d1, b, W, att_src, att_dst):
    h, a_s, a_d, ub = pl.pallas_call(
        _mid_body,
        out_shape=(
            jax.ShapeDtypeStruct((N, EMB), jnp.float32),
            jax.ShapeDtypeStruct((NPAD, 1), jnp.float32),
            jax.ShapeDtypeStruct((NPAD, 1), jnp.float32),
            jax.ShapeDtypeStruct((1, 16), jnp.float32),
        ),
    )(n0, n1, d0.reshape(N, 1), d1.reshape(N, 1), b.reshape(1, EMB), W,
      att_src.reshape(EMB, 1), att_dst.reshape(EMB, 1))
    return h, a_s[:, 0], a_d[:, 0], ub.reshape(16)


def _fin_body(n0_ref, n1_ref, d0_ref, d1_ref, b_ref, lw_ref, lb_ref,
              res_ref, rm_ref, x2_ref, sm_ref, loss_ref):
    inv = 1.0 / (d0_ref[:] + d1_ref[:])
    x2 = jax.nn.relu((n0_ref[:] + n1_ref[:]) * inv + b_ref[:])
    x2_ref[:] = x2
    li = jnp.dot(x2, lw_ref[:], preferred_element_type=jnp.float32) + lb_ref[0, 0]
    li = jnp.where(rm_ref[:] == 0, -1e9, li)
    m = jnp.max(li)
    ex = jnp.exp(li - m)
    sm = ex / jnp.sum(ex)
    sm_ref[:] = sm
    loss = -jnp.sum(jnp.log(jnp.clip(sm, 1e-10, 1.0)) * res_ref[:])
    loss_ref[:] = jnp.full((1, 1), loss, jnp.float32)


def _tc_fin(n0, n1, d0, d1, b, lin_w, lin_b, res, resmask):
    x2, sm, loss = pl.pallas_call(
        _fin_body,
        out_shape=(
            jax.ShapeDtypeStruct((N, EMB), jnp.float32),
            jax.ShapeDtypeStruct((N, 1), jnp.float32),
            jax.ShapeDtypeStruct((1, 1), jnp.float32),
        ),
    )(n0, n1, d0.reshape(N, 1), d1.reshape(N, 1), b.reshape(1, EMB),
      lin_w, lin_b.reshape(1, 1), res.reshape(N, 1), resmask.reshape(N, 1))
    return x2, sm[:, 0], loss.reshape(())


def _sc_body(h_hbm, src_hbm, dst_hbm, as_hbm, ad_hbm, ub_hbm,
             outn_hbm, outd_hbm,
             src_c, dst_c, rows_v, asg_v, adg_v, ex_r, zer_v, ub_v,
             sem1, sem2, sem3,
             acc_s, dcol_s):
    c = lax.axis_index("c")
    s = lax.axis_index("s")
    w = c * 16 + s

    pltpu.sync_copy(ub_hbm, ub_v)

    def _zr(i, _):
        for q in range(8):
            rows_v[i, pl.ds(q * 16, 16)] = jnp.zeros((16,), jnp.float32)
        return 0
    lax.fori_loop(0, 128, _zr, 0)

    def _zd(i, _):
        zer_v[pl.ds(i * 16, 16)] = jnp.zeros((16,), jnp.float32)
        return 0
    lax.fori_loop(0, SL // 16, _zd, 0)

    for t in range(SL // 128):
        pltpu.sync_copy(rows_v, acc_s.at[pl.ds(s * SL + t * 128, 128)])
    pltpu.sync_copy(zer_v, dcol_s.at[pl.ds(s * SL, SL)])
    plsc.subcore_barrier()

    ub16 = ub_v[:]

    def _row(jj):
        cp1 = pltpu.async_copy(as_hbm.at[src_c.at[jj]], asg_v, sem1)
        cp2 = pltpu.async_copy(ad_hbm.at[dst_c.at[jj]], adg_v, sem2)
        cp3 = pltpu.async_copy(h_hbm.at[src_c.at[jj]], rows_v, sem3)
        cp1.wait()
        cp2.wait()
        for q in range(8):
            e = asg_v[pl.ds(q * 16, 16)] + adg_v[pl.ds(q * 16, 16)]
            e = jnp.where(e < 0, NEG_SLOPE * e, e) - ub16
            ex_r[pl.ds(q * 16, 16)] = jnp.exp(e)
        cp3.wait()

        def _scale(r, _):
            ev = plsc.load_gather(ex_r, [jnp.full((16,), r, jnp.int32)])
            for q in range(8):
                rows_v[r, pl.ds(q * 16, 16)] = rows_v[r, pl.ds(q * 16, 16)] * ev
            return 0
        lax.fori_loop(0, 128, _scale, 0)

        pltpu.sync_copy(rows_v, acc_s.at[dst_c.at[jj]], add=True)
        pltpu.sync_copy(ex_r, dcol_s.at[dst_c.at[jj]], add=True)

    def _chunk(k, _):
        off = pl.multiple_of(k * 16, 16)
        pltpu.sync_copy(src_hbm.at[w, pl.ds(off, 16)], src_c)
        pltpu.sync_copy(dst_hbm.at[w, pl.ds(off, 16)], dst_c)

        def _j(jj, _):
            _row(jj)
            return 0
        lax.fori_loop(0, 16, _j, 0)
        return 0
    lax.fori_loop(0, TROWS // 16, _chunk, 0)

    pltpu.sync_copy(src_hbm.at[w, pl.ds(16 * (TROWS // 16), 1)],
                    src_c.at[pl.ds(0, 1)])
    pltpu.sync_copy(dst_hbm.at[w, pl.ds(16 * (TROWS // 16), 1)],
                    dst_c.at[pl.ds(0, 1)])
    _row(0)
    plsc.subcore_barrier()

    pltpu.sync_copy(acc_s.at[pl.ds(s * SL, SL)],
                    outn_hbm.at[pl.ds(c * NPAD + s * SL, SL)])
    pltpu.sync_copy(dcol_s.at[pl.ds(s * SL, SL)],
                    outd_hbm.at[pl.ds(c * NPAD + s * SL, SL)])


@functools.partial(jax.jit, static_argnames=())
def _sc_edge(h, src2, dst2, a_s, a_d, ub):
    mesh = plsc.VectorSubcoreMesh(core_axis_name="c", subcore_axis_name="s")
    k = pl.kernel(
        _sc_body,
        out_type=(
            jax.ShapeDtypeStruct((2 * NPAD, EMB), jnp.float32),
            jax.ShapeDtypeStruct((2 * NPAD,), jnp.float32),
        ),
        mesh=mesh,
        compiler_params=pltpu.CompilerParams(needs_layout_passes=False),
        scratch_types=[
            pltpu.VMEM((16, 128), jnp.int32),
            pltpu.VMEM((16, 128), jnp.int32),
            pltpu.VMEM((128, EMB), jnp.float32),
            pltpu.VMEM((128,), jnp.float32),
            pltpu.VMEM((128,), jnp.float32),
            pltpu.VMEM((128,), jnp.float32),
            pltpu.VMEM((SL,), jnp.float32),
            pltpu.VMEM((16,), jnp.float32),
            pltpu.SemaphoreType.DMA,
            pltpu.SemaphoreType.DMA,
            pltpu.SemaphoreType.DMA,
            pltpu.VMEM_SHARED((NPAD, EMB), jnp.float32),
            pltpu.VMEM_SHARED((NPAD,), jnp.float32),
        ],
    )
    return k(h, src2, dst2, a_s, a_d, ub)


def kernel(input_node, inputad, res, resmask, W1, att_src1, att_dst1, b1,
           W2, att_src2, att_dst2, b2, lin_w, lin_b):
    npad = EP - (inputad.shape[1] + N)
    loop = jnp.arange(N, dtype=jnp.int32)
    padi = jnp.arange(npad, dtype=jnp.int32)
    src = jnp.concatenate([inputad[0], loop, padi % 128])
    dst = jnp.concatenate([inputad[1], loop, N + padi % (NPAD - N)])
    src2 = src.reshape(32, TROWS, 128)
    dst2 = dst.reshape(32, TROWS, 128)

    h1, as1, ad1, ub1 = _tc_pre(input_node, W1, att_src1, att_dst1)
    n1a, d1a = _sc_edge(h1, src2, dst2, as1, ad1, ub1)
    h2, as2, ad2, ub2 = _tc_mid(n1a[:N], n1a[NPAD:NPAD + N],
                                d1a[:N], d1a[NPAD:NPAD + N],
                                b1, W2, att_src2, att_dst2)
    n2a, d2a = _sc_edge(h2, src2, dst2, as2, ad2, ub2)
    x2, sm, loss = _tc_fin(n2a[:N], n2a[NPAD:NPAD + N],
                           d2a[:N], d2a[NPAD:NPAD + N],
                           b2, lin_w, lin_b, res, resmask)
    return (loss, sm, x2, resmask, x2)


# speedup vs baseline: 45.2922x; 1.5044x over previous
import functools

import jax
import jax.numpy as jnp
from jax import lax
from jax.experimental import pallas as pl
from jax.experimental.pallas import tpu as pltpu
from jax.experimental.pallas import tpu_sc as plsc

N = 10000
EMB = 128
NEG_SLOPE = 0.2
NPAD = 10240
EP = 331776
EROWS = EP // 128
TROWS = EROWS // 32
SL = NPAD // 16


def _proj(h, asv_ref, adv_ref):
    a_s = jnp.dot(h, asv_ref[:], preferred_element_type=jnp.float32)
    a_d = jnp.dot(h, adv_ref[:], preferred_element_type=jnp.float32)
    pad = jnp.zeros((NPAD - N, 1), jnp.float32)
    ub = jnp.max(a_s) + jnp.max(a_d)
    ub = jnp.where(ub < 0, NEG_SLOPE * ub, ub)
    return (jnp.concatenate([a_s, pad], axis=0),
            jnp.concatenate([a_d, pad], axis=0),
            jnp.full((1, 16), ub, jnp.float32))


def _pre_body(x_ref, w_ref, asv_ref, adv_ref, h_ref, as_ref, ad_ref, ub_ref):
    h = jnp.dot(x_ref[:], w_ref[:], preferred_element_type=jnp.float32)
    h_ref[:] = h
    as_ref[:], ad_ref[:], ub_ref[:] = _proj(h, asv_ref, adv_ref)


def _tc_pre(x, W, att_src, att_dst):
    h, a_s, a_d, ub = pl.pallas_call(
        _pre_body,
        out_shape=(
            jax.ShapeDtypeStruct((N, EMB), jnp.float32),
            jax.ShapeDtypeStruct((NPAD, 1), jnp.float32),
            jax.ShapeDtypeStruct((NPAD, 1), jnp.float32),
            jax.ShapeDtypeStruct((1, 16), jnp.float32),
        ),
    )(x, W, att_src.reshape(EMB, 1), att_dst.reshape(EMB, 1))
    return h, a_s[:, 0], a_d[:, 0], ub.reshape(16)


def _mid_body(n0_ref, n1_ref, d0_ref, d1_ref, b_ref, w_ref, asv_ref, adv_ref,
              h_ref, as_ref, ad_ref, ub_ref):
    inv = 1.0 / (d0_ref[:] + d1_ref[:])
    x = jax.nn.relu((n0_ref[:] + n1_ref[:]) * inv + b_ref[:])
    h = jnp.dot(x, w_ref[:], preferred_element_type=jnp.float32)
    h_ref[:] = h
    as_ref[:], ad_ref[:], ub_ref[:] = _proj(h, asv_ref, adv_ref)


def _tc_mid(n0, n1, d0, d1, b, W, att_src, att_dst):
    h, a_s, a_d, ub = pl.pallas_call(
        _mid_body,
        out_shape=(
            jax.ShapeDtypeStruct((N, EMB), jnp.float32),
            jax.ShapeDtypeStruct((NPAD, 1), jnp.float32),
            jax.ShapeDtypeStruct((NPAD, 1), jnp.float32),
            jax.ShapeDtypeStruct((1, 16), jnp.float32),
        ),
    )(n0, n1, d0.reshape(N, 1), d1.reshape(N, 1), b.reshape(1, EMB), W,
      att_src.reshape(EMB, 1), att_dst.reshape(EMB, 1))
    return h, a_s[:, 0], a_d[:, 0], ub.reshape(16)


def _fin_body(n0_ref, n1_ref, d0_ref, d1_ref, b_ref, lw_ref, lb_ref,
              res_ref, rm_ref, x2_ref, sm_ref, loss_ref):
    inv = 1.0 / (d0_ref[:] + d1_ref[:])
    x2 = jax.nn.relu((n0_ref[:] + n1_ref[:]) * inv + b_ref[:])
    x2_ref[:] = x2
    li = jnp.dot(x2, lw_ref[:], preferred_element_type=jnp.float32) + lb_ref[0, 0]
    li = jnp.where(rm_ref[:] == 0, -1e9, li)
    m = jnp.max(li)
    ex = jnp.exp(li - m)
    sm = ex / jnp.sum(ex)
    sm_ref[:] = sm
    loss = -jnp.sum(jnp.log(jnp.clip(sm, 1e-10, 1.0)) * res_ref[:])
    loss_ref[:] = jnp.full((1, 1), loss, jnp.float32)


def _tc_fin(n0, n1, d0, d1, b, lin_w, lin_b, res, resmask):
    x2, sm, loss = pl.pallas_call(
        _fin_body,
        out_shape=(
            jax.ShapeDtypeStruct((N, EMB), jnp.float32),
            jax.ShapeDtypeStruct((N, 1), jnp.float32),
            jax.ShapeDtypeStruct((1, 1), jnp.float32),
        ),
    )(n0, n1, d0.reshape(N, 1), d1.reshape(N, 1), b.reshape(1, EMB),
      lin_w, lin_b.reshape(1, 1), res.reshape(N, 1), resmask.reshape(N, 1))
    return x2, sm[:, 0], loss.reshape(())


def _sc_body(h_hbm, src_hbm, dst_hbm, as_hbm, ad_hbm, ub_hbm,
             outn_hbm, outd_hbm,
             src_c0, dst_c0, src_c1, dst_c1, src_t, dst_t,
             rows0, rows1, asg0, adg0, ex0, asg1, adg1, ex1, zer_v, ub_v,
             sas0, sad0, srw0, ssc0, sas1, sad1, srw1, ssc1,
             acc_s, dcol_s):
    c = lax.axis_index("c")
    s = lax.axis_index("s")
    w = c * 16 + s

    pltpu.sync_copy(ub_hbm, ub_v)

    def _zr(i, _):
        for q in range(8):
            rows0[i, pl.ds(q * 16, 16)] = jnp.zeros((16,), jnp.float32)
            rows1[i, pl.ds(q * 16, 16)] = jnp.zeros((16,), jnp.float32)
        return 0
    lax.fori_loop(0, 128, _zr, 0)

    def _zd(i, _):
        zer_v[pl.ds(i * 16, 16)] = jnp.zeros((16,), jnp.float32)
        return 0
    lax.fori_loop(0, SL // 16, _zd, 0)

    for t in range(SL // 128):
        pltpu.sync_copy(rows0, acc_s.at[pl.ds(s * SL + t * 128, 128)])
    pltpu.sync_copy(zer_v, dcol_s.at[pl.ds(s * SL, SL)])
    plsc.subcore_barrier()

    ub16 = ub_v[:]

    B0 = (rows0, asg0, adg0, ex0, sas0, sad0, srw0, ssc0)
    B1 = (rows1, asg1, adg1, ex1, sas1, sad1, srw1, ssc1)
    C0 = (src_c0, dst_c0)
    C1 = (src_c1, dst_c1)
    CT = (src_t, dst_t)

    def _issue(C, rr, B):
        rows, asg, adg, ex, sas, sad, srw, ssc = B
        srcr, dstr = C
        pltpu.make_async_copy(rows, acc_s.at[dstr.at[rr]], ssc).wait()
        pltpu.async_copy(as_hbm.at[srcr.at[rr]], asg, sas)
        pltpu.async_copy(ad_hbm.at[dstr.at[rr]], adg, sad)
        pltpu.async_copy(h_hbm.at[srcr.at[rr]], rows, srw)

    def _process(C, rr, B):
        rows, asg, adg, ex, sas, sad, srw, ssc = B
        srcr, dstr = C
        pltpu.make_async_copy(as_hbm.at[srcr.at[rr]], asg, sas).wait()
        pltpu.make_async_copy(ad_hbm.at[dstr.at[rr]], adg, sad).wait()
        for q in range(8):
            e = asg[pl.ds(q * 16, 16)] + adg[pl.ds(q * 16, 16)]
            e = jnp.where(e < 0, NEG_SLOPE * e, e) - ub16
            ex[pl.ds(q * 16, 16)] = jnp.exp(e)
        pltpu.make_async_copy(h_hbm.at[srcr.at[rr]], rows, srw).wait()

        def _scale(r2, _):
            for u in range(2):
                r = r2 * 2 + u
                ev = plsc.load_gather(ex, [jnp.full((16,), r, jnp.int32)])
                for q in range(8):
                    rows[r, pl.ds(q * 16, 16)] = rows[r, pl.ds(q * 16, 16)] * ev
            return 0
        lax.fori_loop(0, 64, _scale, 0)

        pltpu.async_copy(rows, acc_s.at[dstr.at[rr]], ssc, add=True)
        pltpu.sync_copy(ex, dcol_s.at[dstr.at[rr]], add=True)

    def _stage(ck, C):
        srcr, dstr = C
        off = ck * 16
        pltpu.sync_copy(src_hbm.at[w, pl.ds(off, 16)], srcr)
        pltpu.sync_copy(dst_hbm.at[w, pl.ds(off, 16)], dstr)

    _stage(0, C0)
    pltpu.sync_copy(src_hbm.at[w, pl.ds(80, 1)], src_t)
    pltpu.sync_copy(dst_hbm.at[w, pl.ds(80, 1)], dst_t)
    pltpu.async_copy(rows0, acc_s.at[dst_c0.at[0]], ssc0, add=True)
    pltpu.async_copy(rows1, acc_s.at[dst_c0.at[0]], ssc1, add=True)
    _issue(C0, 0, B0)

    for ck in range(5):
        C = C0 if ck % 2 == 0 else C1
        Cn = C1 if ck % 2 == 0 else C0

        def _pair(p, _, C=C, Cn=Cn, ck=ck):
            a = 2 * p
            _issue(C, a + 1, B1)
            _process(C, a, B0)

            @pl.when(p < 7)
            def _():
                _issue(C, a + 2, B0)

            @pl.when(p == 7)
            def _():
                if ck < 4:
                    _stage(ck + 1, Cn)
                    _issue(Cn, 0, B0)
                else:
                    _issue(CT, 0, B0)

            _process(C, a + 1, B1)
            return 0
        lax.fori_loop(0, 8, _pair, 0)

    _process(CT, 0, B0)
    pltpu.make_async_copy(rows0, acc_s.at[dst_t.at[0]], ssc0).wait()
    pltpu.make_async_copy(rows1, acc_s.at[dst_t.at[0]], ssc1).wait()
    plsc.subcore_barrier()

    pltpu.sync_copy(acc_s.at[pl.ds(s * SL, SL)],
                    outn_hbm.at[pl.ds(c * NPAD + s * SL, SL)])
    pltpu.sync_copy(dcol_s.at[pl.ds(s * SL, SL)],
                    outd_hbm.at[pl.ds(c * NPAD + s * SL, SL)])


@functools.partial(jax.jit, static_argnames=())
def _sc_edge(h, src2, dst2, a_s, a_d, ub):
    mesh = plsc.VectorSubcoreMesh(core_axis_name="c", subcore_axis_name="s")
    k = pl.kernel(
        _sc_body,
        out_type=(
            jax.ShapeDtypeStruct((2 * NPAD, EMB), jnp.float32),
            jax.ShapeDtypeStruct((2 * NPAD,), jnp.float32),
        ),
        mesh=mesh,
        compiler_params=pltpu.CompilerParams(needs_layout_passes=False),
        scratch_types=[
            pltpu.VMEM((16, 128), jnp.int32),
            pltpu.VMEM((16, 128), jnp.int32),
            pltpu.VMEM((16, 128), jnp.int32),
            pltpu.VMEM((16, 128), jnp.int32),
            pltpu.VMEM((1, 128), jnp.int32),
            pltpu.VMEM((1, 128), jnp.int32),
            pltpu.VMEM((128, EMB), jnp.float32),
            pltpu.VMEM((128, EMB), jnp.float32),
            pltpu.VMEM((128,), jnp.float32),
            pltpu.VMEM((128,), jnp.float32),
            pltpu.VMEM((128,), jnp.float32),
            pltpu.VMEM((128,), jnp.float32),
            pltpu.VMEM((128,), jnp.float32),
            pltpu.VMEM((128,), jnp.float32),
            pltpu.VMEM((SL,), jnp.float32),
            pltpu.VMEM((16,), jnp.float32),
            pltpu.SemaphoreType.DMA,
            pltpu.SemaphoreType.DMA,
            pltpu.SemaphoreType.DMA,
            pltpu.SemaphoreType.DMA,
            pltpu.SemaphoreType.DMA,
            pltpu.SemaphoreType.DMA,
            pltpu.SemaphoreType.DMA,
            pltpu.SemaphoreType.DMA,
            pltpu.VMEM_SHARED((NPAD, EMB), jnp.float32),
            pltpu.VMEM_SHARED((NPAD,), jnp.float32),
        ],
    )
    return k(h, src2, dst2, a_s, a_d, ub)


def kernel(input_node, inputad, res, resmask, W1, att_src1, att_dst1, b1,
           W2, att_src2, att_dst2, b2, lin_w, lin_b):
    npad = EP - (inputad.shape[1] + N)
    loop = jnp.arange(N, dtype=jnp.int32)
    padi = jnp.arange(npad, dtype=jnp.int32)
    src = jnp.concatenate([inputad[0], loop, padi % 128])
    dst = jnp.concatenate([inputad[1], loop, N + padi % (NPAD - N)])
    src2 = src.reshape(32, TROWS, 128)
    dst2 = dst.reshape(32, TROWS, 128)

    h1, as1, ad1, ub1 = _tc_pre(input_node, W1, att_src1, att_dst1)
    n1a, d1a = _sc_edge(h1, src2, dst2, as1, ad1, ub1)
    h2, as2, ad2, ub2 = _tc_mid(n1a[:N], n1a[NPAD:NPAD + N],
                                d1a[:N], d1a[NPAD:NPAD + N],
                                b1, W2, att_src2, att_dst2)
    n2a, d2a = _sc_edge(h2, src2, dst2, as2, ad2, ub2)
    x2, sm, loss = _tc_fin(n2a[:N], n2a[NPAD:NPAD + N],
                           d2a[:N], d2a[NPAD:NPAD + N],
                           b2, lin_w, lin_b, res, resmask)
    return (loss, sm, x2, resmask, x2)


# speedup vs baseline: 47.4579x; 1.0478x over previous
import functools

import jax
import jax.numpy as jnp
from jax import lax
from jax.experimental import pallas as pl
from jax.experimental.pallas import tpu as pltpu
from jax.experimental.pallas import tpu_sc as plsc

N = 10000
EMB = 128
NEG_SLOPE = 0.2
NPAD = 10240
EP = 331776
EROWS = EP // 128
TROWS = EROWS // 32
SL = NPAD // 16


def _proj(h, asv_ref, adv_ref):
    a_s = jnp.dot(h, asv_ref[:], preferred_element_type=jnp.float32)
    a_d = jnp.dot(h, adv_ref[:], preferred_element_type=jnp.float32)
    pad = jnp.zeros((NPAD - N, 1), jnp.float32)
    ub = jnp.max(a_s) + jnp.max(a_d)
    ub = jnp.where(ub < 0, NEG_SLOPE * ub, ub)
    return (jnp.concatenate([a_s, pad], axis=0),
            jnp.concatenate([a_d, pad], axis=0),
            jnp.full((1, 16), ub, jnp.float32))


def _pre_body(x_ref, w_ref, asv_ref, adv_ref, h_ref, as_ref, ad_ref, ub_ref):
    h = jnp.dot(x_ref[:], w_ref[:], preferred_element_type=jnp.float32)
    h_ref[:] = h
    as_ref[:], ad_ref[:], ub_ref[:] = _proj(h, asv_ref, adv_ref)


def _tc_pre(x, W, att_src, att_dst):
    h, a_s, a_d, ub = pl.pallas_call(
        _pre_body,
        out_shape=(
            jax.ShapeDtypeStruct((N, EMB), jnp.float32),
            jax.ShapeDtypeStruct((NPAD, 1), jnp.float32),
            jax.ShapeDtypeStruct((NPAD, 1), jnp.float32),
            jax.ShapeDtypeStruct((1, 16), jnp.float32),
        ),
    )(x, W, att_src.reshape(EMB, 1), att_dst.reshape(EMB, 1))
    return h, a_s[:, 0], a_d[:, 0], ub.reshape(16)


def _mid_body(n_ref, d_ref, b_ref, w_ref, asv_ref, adv_ref,
              h_ref, as_ref, ad_ref, ub_ref):
    inv = 1.0 / (d_ref[0:N] + d_ref[NPAD:NPAD + N])
    x = jax.nn.relu((n_ref[0:N] + n_ref[NPAD:NPAD + N]) * inv + b_ref[:])
    h = jnp.dot(x, w_ref[:], preferred_element_type=jnp.float32)
    h_ref[:] = h
    as_ref[:], ad_ref[:], ub_ref[:] = _proj(h, asv_ref, adv_ref)


def _tc_mid(n, d, b, W, att_src, att_dst):
    h, a_s, a_d, ub = pl.pallas_call(
        _mid_body,
        out_shape=(
            jax.ShapeDtypeStruct((N, EMB), jnp.float32),
            jax.ShapeDtypeStruct((NPAD, 1), jnp.float32),
            jax.ShapeDtypeStruct((NPAD, 1), jnp.float32),
            jax.ShapeDtypeStruct((1, 16), jnp.float32),
        ),
    )(n, d.reshape(2 * NPAD, 1), b.reshape(1, EMB), W,
      att_src.reshape(EMB, 1), att_dst.reshape(EMB, 1))
    return h, a_s[:, 0], a_d[:, 0], ub.reshape(16)


def _fin_body(n_ref, d_ref, b_ref, lw_ref, lb_ref,
              res_ref, rm_ref, x2_ref, sm_ref, loss_ref):
    inv = 1.0 / (d_ref[0:N] + d_ref[NPAD:NPAD + N])
    x2 = jax.nn.relu((n_ref[0:N] + n_ref[NPAD:NPAD + N]) * inv + b_ref[:])
    x2_ref[:] = x2
    li = jnp.dot(x2, lw_ref[:], preferred_element_type=jnp.float32) + lb_ref[0, 0]
    li = jnp.where(rm_ref[:] == 0, -1e9, li)
    m = jnp.max(li)
    ex = jnp.exp(li - m)
    sm = ex / jnp.sum(ex)
    sm_ref[:] = sm
    loss = -jnp.sum(jnp.log(jnp.clip(sm, 1e-10, 1.0)) * res_ref[:])
    loss_ref[:] = jnp.full((1, 1), loss, jnp.float32)


def _tc_fin(n, d, b, lin_w, lin_b, res, resmask):
    x2, sm, loss = pl.pallas_call(
        _fin_body,
        out_shape=(
            jax.ShapeDtypeStruct((N, EMB), jnp.float32),
            jax.ShapeDtypeStruct((N, 1), jnp.float32),
            jax.ShapeDtypeStruct((1, 1), jnp.float32),
        ),
    )(n, d.reshape(2 * NPAD, 1), b.reshape(1, EMB),
      lin_w, lin_b.reshape(1, 1), res.reshape(N, 1), resmask.reshape(N, 1))
    return x2, sm[:, 0], loss.reshape(())


def _sc_body(h_hbm, src_hbm, dst_hbm, as_hbm, ad_hbm, ub_hbm,
             outn_hbm, outd_hbm,
             src_c0, dst_c0, src_c1, dst_c1, src_t, dst_t,
             rows0, rows1, asg0, adg0, ex0, asg1, adg1, ex1, zer_v, ub_v,
             sas0, sad0, srw0, ssc0, sdc0, sas1, sad1, srw1, ssc1, sdc1,
             acc_s, dcol_s):
    c = lax.axis_index("c")
    s = lax.axis_index("s")
    w = c * 16 + s

    pltpu.sync_copy(ub_hbm, ub_v)

    def _zr(i, _):
        for q in range(8):
            rows0[i, pl.ds(q * 16, 16)] = jnp.zeros((16,), jnp.float32)
            rows1[i, pl.ds(q * 16, 16)] = jnp.zeros((16,), jnp.float32)
        return 0
    lax.fori_loop(0, 128, _zr, 0)
    for q in range(8):
        ex0[pl.ds(q * 16, 16)] = jnp.zeros((16,), jnp.float32)
        ex1[pl.ds(q * 16, 16)] = jnp.zeros((16,), jnp.float32)

    def _zd(i, _):
        zer_v[pl.ds(i * 16, 16)] = jnp.zeros((16,), jnp.float32)
        return 0
    lax.fori_loop(0, SL // 16, _zd, 0)

    for t in range(SL // 128):
        pltpu.sync_copy(rows0, acc_s.at[pl.ds(s * SL + t * 128, 128)])
    pltpu.sync_copy(zer_v, dcol_s.at[pl.ds(s * SL, SL)])
    plsc.subcore_barrier()

    ub16 = ub_v[:]

    B0 = (rows0, asg0, adg0, ex0, sas0, sad0, srw0, ssc0, sdc0)
    B1 = (rows1, asg1, adg1, ex1, sas1, sad1, srw1, ssc1, sdc1)
    C0 = (src_c0, dst_c0)
    C1 = (src_c1, dst_c1)
    CT = (src_t, dst_t)

    def _issue(C, rr, B):
        rows, asg, adg, ex, sas, sad, srw, ssc, sdc = B
        srcr, dstr = C
        pltpu.make_async_copy(rows, acc_s.at[dstr.at[rr]], ssc).wait()
        pltpu.make_async_copy(ex, dcol_s.at[dstr.at[rr]], sdc).wait()
        pltpu.async_copy(as_hbm.at[srcr.at[rr]], asg, sas)
        pltpu.async_copy(ad_hbm.at[dstr.at[rr]], adg, sad)
        pltpu.async_copy(h_hbm.at[srcr.at[rr]], rows, srw)

    def _process(C, rr, B):
        rows, asg, adg, ex, sas, sad, srw, ssc, sdc = B
        srcr, dstr = C
        pltpu.make_async_copy(as_hbm.at[srcr.at[rr]], asg, sas).wait()
        pltpu.make_async_copy(ad_hbm.at[dstr.at[rr]], adg, sad).wait()
        for q in range(8):
            e = asg[pl.ds(q * 16, 16)] + adg[pl.ds(q * 16, 16)]
            e = jnp.where(e < 0, NEG_SLOPE * e, e) - ub16
            ex[pl.ds(q * 16, 16)] = jnp.exp(e)
        pltpu.make_async_copy(h_hbm.at[srcr.at[rr]], rows, srw).wait()

        def _scale(r4, _):
            for u in range(4):
                r = r4 * 4 + u
                ev = plsc.load_gather(ex, [jnp.full((16,), r, jnp.int32)])
                for q in range(8):
                    rows[r, pl.ds(q * 16, 16)] = rows[r, pl.ds(q * 16, 16)] * ev
            return 0
        lax.fori_loop(0, 32, _scale, 0)

        pltpu.async_copy(rows, acc_s.at[dstr.at[rr]], ssc, add=True)
        pltpu.async_copy(ex, dcol_s.at[dstr.at[rr]], sdc, add=True)

    def _stage(ck, C):
        srcr, dstr = C
        off = ck * 16
        pltpu.sync_copy(src_hbm.at[w, pl.ds(off, 16)], srcr)
        pltpu.sync_copy(dst_hbm.at[w, pl.ds(off, 16)], dstr)

    _stage(0, C0)
    pltpu.sync_copy(src_hbm.at[w, pl.ds(80, 1)], src_t)
    pltpu.sync_copy(dst_hbm.at[w, pl.ds(80, 1)], dst_t)
    pltpu.async_copy(rows0, acc_s.at[dst_c0.at[0]], ssc0, add=True)
    pltpu.async_copy(rows1, acc_s.at[dst_c0.at[0]], ssc1, add=True)
    pltpu.async_copy(ex0, dcol_s.at[dst_c0.at[0]], sdc0, add=True)
    pltpu.async_copy(ex1, dcol_s.at[dst_c0.at[0]], sdc1, add=True)
    _issue(C0, 0, B0)

    for ck in range(5):
        C = C0 if ck % 2 == 0 else C1
        Cn = C1 if ck % 2 == 0 else C0

        def _pair(p, _, C=C, Cn=Cn, ck=ck):
            a = 2 * p
            _issue(C, a + 1, B1)
            _process(C, a, B0)

            @pl.when(p < 7)
            def _():
                _issue(C, a + 2, B0)

            @pl.when(p == 7)
            def _():
                if ck < 4:
                    _stage(ck + 1, Cn)
                    _issue(Cn, 0, B0)
                else:
                    _issue(CT, 0, B0)

            _process(C, a + 1, B1)
            return 0
        lax.fori_loop(0, 8, _pair, 0)

    _process(CT, 0, B0)
    pltpu.make_async_copy(rows0, acc_s.at[dst_t.at[0]], ssc0).wait()
    pltpu.make_async_copy(rows1, acc_s.at[dst_t.at[0]], ssc1).wait()
    pltpu.make_async_copy(ex0, dcol_s.at[dst_t.at[0]], sdc0).wait()
    pltpu.make_async_copy(ex1, dcol_s.at[dst_t.at[0]], sdc1).wait()
    plsc.subcore_barrier()

    pltpu.sync_copy(acc_s.at[pl.ds(s * SL, SL)],
                    outn_hbm.at[pl.ds(c * NPAD + s * SL, SL)])
    pltpu.sync_copy(dcol_s.at[pl.ds(s * SL, SL)],
                    outd_hbm.at[pl.ds(c * NPAD + s * SL, SL)])


@functools.partial(jax.jit, static_argnames=())
def _sc_edge(h, src2, dst2, a_s, a_d, ub):
    mesh = plsc.VectorSubcoreMesh(core_axis_name="c", subcore_axis_name="s")
    k = pl.kernel(
        _sc_body,
        out_type=(
            jax.ShapeDtypeStruct((2 * NPAD, EMB), jnp.float32),
            jax.ShapeDtypeStruct((2 * NPAD,), jnp.float32),
        ),
        mesh=mesh,
        compiler_params=pltpu.CompilerParams(needs_layout_passes=False),
        scratch_types=[
            pltpu.VMEM((16, 128), jnp.int32),
            pltpu.VMEM((16, 128), jnp.int32),
            pltpu.VMEM((16, 128), jnp.int32),
            pltpu.VMEM((16, 128), jnp.int32),
            pltpu.VMEM((1, 128), jnp.int32),
            pltpu.VMEM((1, 128), jnp.int32),
            pltpu.VMEM((128, EMB), jnp.float32),
            pltpu.VMEM((128, EMB), jnp.float32),
            pltpu.VMEM((128,), jnp.float32),
            pltpu.VMEM((128,), jnp.float32),
            pltpu.VMEM((128,), jnp.float32),
            pltpu.VMEM((128,), jnp.float32),
            pltpu.VMEM((128,), jnp.float32),
            pltpu.VMEM((128,), jnp.float32),
            pltpu.VMEM((SL,), jnp.float32),
            pltpu.VMEM((16,), jnp.float32),
            pltpu.SemaphoreType.DMA,
            pltpu.SemaphoreType.DMA,
            pltpu.SemaphoreType.DMA,
            pltpu.SemaphoreType.DMA,
            pltpu.SemaphoreType.DMA,
            pltpu.SemaphoreType.DMA,
            pltpu.SemaphoreType.DMA,
            pltpu.SemaphoreType.DMA,
            pltpu.SemaphoreType.DMA,
            pltpu.SemaphoreType.DMA,
            pltpu.VMEM_SHARED((NPAD, EMB), jnp.float32),
            pltpu.VMEM_SHARED((NPAD,), jnp.float32),
        ],
    )
    return k(h, src2, dst2, a_s, a_d, ub)


def kernel(input_node, inputad, res, resmask, W1, att_src1, att_dst1, b1,
           W2, att_src2, att_dst2, b2, lin_w, lin_b):
    npad = EP - (inputad.shape[1] + N)
    loop = jnp.arange(N, dtype=jnp.int32)
    padi = jnp.arange(npad, dtype=jnp.int32)
    src = jnp.concatenate([inputad[0], loop, padi % 128])
    dst = jnp.concatenate([inputad[1], loop, N + padi % (NPAD - N)])
    src2 = src.reshape(32, TROWS, 128)
    dst2 = dst.reshape(32, TROWS, 128)

    h1, as1, ad1, ub1 = _tc_pre(input_node, W1, att_src1, att_dst1)
    n1a, d1a = _sc_edge(h1, src2, dst2, as1, ad1, ub1)
    h2, as2, ad2, ub2 = _tc_mid(n1a, d1a, b1, W2, att_src2, att_dst2)
    n2a, d2a = _sc_edge(h2, src2, dst2, as2, ad2, ub2)
    x2, sm, loss = _tc_fin(n2a, d2a, b2, lin_w, lin_b, res, resmask)
    return (loss, sm, x2, resmask, x2)


# speedup vs baseline: 49.2132x; 1.0370x over previous
import functools

import jax
import jax.numpy as jnp
from jax import lax
from jax.experimental import pallas as pl
from jax.experimental.pallas import tpu as pltpu
from jax.experimental.pallas import tpu_sc as plsc

N = 10000
EMB = 128
NEG_SLOPE = 0.2
NPAD = 10240
EP = 331776
CR = 64
TBLK = EP // (32 * CR)
SL = NPAD // 16


def _proj(h, asv_ref, adv_ref):
    a_s = jnp.dot(h, asv_ref[:], preferred_element_type=jnp.float32)
    a_d = jnp.dot(h, adv_ref[:], preferred_element_type=jnp.float32)
    pad = jnp.zeros((NPAD - N, 1), jnp.float32)
    ub = jnp.max(a_s) + jnp.max(a_d)
    ub = jnp.where(ub < 0, NEG_SLOPE * ub, ub)
    return (jnp.concatenate([a_s, pad], axis=0),
            jnp.concatenate([a_d, pad], axis=0),
            jnp.full((1, 16), ub, jnp.float32))


def _pre_body(x_ref, w_ref, asv_ref, adv_ref, h_ref, as_ref, ad_ref, ub_ref):
    h = jnp.dot(x_ref[:], w_ref[:], preferred_element_type=jnp.float32)
    h_ref[:] = h
    as_ref[:], ad_ref[:], ub_ref[:] = _proj(h, asv_ref, adv_ref)


def _tc_pre(x, W, att_src, att_dst):
    h, a_s, a_d, ub = pl.pallas_call(
        _pre_body,
        out_shape=(
            jax.ShapeDtypeStruct((N, EMB), jnp.float32),
            jax.ShapeDtypeStruct((NPAD, 1), jnp.float32),
            jax.ShapeDtypeStruct((NPAD, 1), jnp.float32),
            jax.ShapeDtypeStruct((1, 16), jnp.float32),
        ),
    )(x, W, att_src.reshape(EMB, 1), att_dst.reshape(EMB, 1))
    return h, a_s[:, 0], a_d[:, 0], ub.reshape(16)


def _mid_body(n_ref, d_ref, b_ref, w_ref, asv_ref, adv_ref,
              h_ref, as_ref, ad_ref, ub_ref):
    inv = 1.0 / (d_ref[0:N] + d_ref[NPAD:NPAD + N])
    x = jax.nn.relu((n_ref[0:N] + n_ref[NPAD:NPAD + N]) * inv + b_ref[:])
    h = jnp.dot(x, w_ref[:], preferred_element_type=jnp.float32)
    h_ref[:] = h
    as_ref[:], ad_ref[:], ub_ref[:] = _proj(h, asv_ref, adv_ref)


def _tc_mid(n, d, b, W, att_src, att_dst):
    h, a_s, a_d, ub = pl.pallas_call(
        _mid_body,
        out_shape=(
            jax.ShapeDtypeStruct((N, EMB), jnp.float32),
            jax.ShapeDtypeStruct((NPAD, 1), jnp.float32),
            jax.ShapeDtypeStruct((NPAD, 1), jnp.float32),
            jax.ShapeDtypeStruct((1, 16), jnp.float32),
        ),
    )(n, d.reshape(2 * NPAD, 1), b.reshape(1, EMB), W,
      att_src.reshape(EMB, 1), att_dst.reshape(EMB, 1))
    return h, a_s[:, 0], a_d[:, 0], ub.reshape(16)


def _fin_body(n_ref, d_ref, b_ref, lw_ref, lb_ref,
              res_ref, rm_ref, x2_ref, sm_ref, loss_ref):
    inv = 1.0 / (d_ref[0:N] + d_ref[NPAD:NPAD + N])
    x2 = jax.nn.relu((n_ref[0:N] + n_ref[NPAD:NPAD + N]) * inv + b_ref[:])
    x2_ref[:] = x2
    li = jnp.dot(x2, lw_ref[:], preferred_element_type=jnp.float32) + lb_ref[0, 0]
    li = jnp.where(rm_ref[:] == 0, -1e9, li)
    m = jnp.max(li)
    ex = jnp.exp(li - m)
    sm = ex / jnp.sum(ex)
    sm_ref[:] = sm
    loss = -jnp.sum(jnp.log(jnp.clip(sm, 1e-10, 1.0)) * res_ref[:])
    loss_ref[:] = jnp.full((1, 1), loss, jnp.float32)


def _tc_fin(n, d, b, lin_w, lin_b, res, resmask):
    x2, sm, loss = pl.pallas_call(
        _fin_body,
        out_shape=(
            jax.ShapeDtypeStruct((N, EMB), jnp.float32),
            jax.ShapeDtypeStruct((N, 1), jnp.float32),
            jax.ShapeDtypeStruct((1, 1), jnp.float32),
        ),
    )(n, d.reshape(2 * NPAD, 1), b.reshape(1, EMB),
      lin_w, lin_b.reshape(1, 1), res.reshape(N, 1), resmask.reshape(N, 1))
    return x2, sm[:, 0], loss.reshape(())


def _sc_body(h_hbm, src_hbm, dst_hbm, as_hbm, ad_hbm, ub_hbm,
             outn_hbm, outd_hbm,
             src_c0, dst_c0, src_c1, dst_c1,
             rows0, rows1, rows2,
             asg0, adg0, ex0, asg1, adg1, ex1, asg2, adg2, ex2,
             zer_v, ub_v,
             sas0, sad0, srw0, ssc0, sdc0,
             sas1, sad1, srw1, ssc1, sdc1,
             sas2, sad2, srw2, ssc2, sdc2,
             acc_s, dcol_s):
    c = lax.axis_index("c")
    s = lax.axis_index("s")
    w = c * 16 + s

    pltpu.sync_copy(ub_hbm, ub_v)

    def _zr(i, _):
        for q in range(8):
            rows0[i, pl.ds(q * 16, 16)] = jnp.zeros((16,), jnp.float32)
            rows1[i, pl.ds(q * 16, 16)] = jnp.zeros((16,), jnp.float32)
            rows2[i, pl.ds(q * 16, 16)] = jnp.zeros((16,), jnp.float32)
        return 0
    lax.fori_loop(0, CR, _zr, 0)
    for q in range(CR // 16):
        ex0[pl.ds(q * 16, 16)] = jnp.zeros((16,), jnp.float32)
        ex1[pl.ds(q * 16, 16)] = jnp.zeros((16,), jnp.float32)
        ex2[pl.ds(q * 16, 16)] = jnp.zeros((16,), jnp.float32)

    def _zd(i, _):
        zer_v[pl.ds(i * 16, 16)] = jnp.zeros((16,), jnp.float32)
        return 0
    lax.fori_loop(0, SL // 16, _zd, 0)

    for t in range(SL // CR):
        pltpu.sync_copy(rows0, acc_s.at[pl.ds(s * SL + t * CR, CR)])
    pltpu.sync_copy(zer_v, dcol_s.at[pl.ds(s * SL, SL)])
    plsc.subcore_barrier()

    ub16 = ub_v[:]

    B0 = (rows0, asg0, adg0, ex0, sas0, sad0, srw0, ssc0, sdc0)
    B1 = (rows1, asg1, adg1, ex1, sas1, sad1, srw1, ssc1, sdc1)
    B2 = (rows2, asg2, adg2, ex2, sas2, sad2, srw2, ssc2, sdc2)
    C0 = (src_c0, dst_c0)
    C1 = (src_c1, dst_c1)

    def _issue(C, rr, B):
        rows, asg, adg, ex, sas, sad, srw, ssc, sdc = B
        srcr, dstr = C
        pltpu.make_async_copy(rows, acc_s.at[dstr.at[rr]], ssc).wait()
        pltpu.make_async_copy(ex, dcol_s.at[dstr.at[rr]], sdc).wait()
        pltpu.async_copy(as_hbm.at[srcr.at[rr]], asg, sas)
        pltpu.async_copy(ad_hbm.at[dstr.at[rr]], adg, sad)
        pltpu.async_copy(h_hbm.at[srcr.at[rr]], rows, srw)

    def _process(C, rr, B):
        rows, asg, adg, ex, sas, sad, srw, ssc, sdc = B
        srcr, dstr = C
        pltpu.make_async_copy(as_hbm.at[srcr.at[rr]], asg, sas).wait()
        pltpu.make_async_copy(ad_hbm.at[dstr.at[rr]], adg, sad).wait()
        for q in range(CR // 16):
            e = asg[pl.ds(q * 16, 16)] + adg[pl.ds(q * 16, 16)]
            e = jnp.where(e < 0, NEG_SLOPE * e, e) - ub16
            ex[pl.ds(q * 16, 16)] = jnp.exp(e)
        pltpu.make_async_copy(h_hbm.at[srcr.at[rr]], rows, srw).wait()

        def _scale(r4, _):
            for u in range(4):
                r = r4 * 4 + u
                ev = plsc.load_gather(ex, [jnp.full((16,), r, jnp.int32)])
                for q in range(8):
                    rows[r, pl.ds(q * 16, 16)] = rows[r, pl.ds(q * 16, 16)] * ev
            return 0
        lax.fori_loop(0, CR // 4, _scale, 0)

        pltpu.async_copy(rows, acc_s.at[dstr.at[rr]], ssc, add=True)
        pltpu.async_copy(ex, dcol_s.at[dstr.at[rr]], sdc, add=True)

    def _stage(ck, C):
        srcr, dstr = C
        nr = 24 if ck < 6 else 18
        off = ck * 24
        if nr == 24:
            pltpu.sync_copy(src_hbm.at[w, pl.ds(off, nr)], srcr)
            pltpu.sync_copy(dst_hbm.at[w, pl.ds(off, nr)], dstr)
        else:
            pltpu.sync_copy(src_hbm.at[w, pl.ds(off, nr)], srcr.at[pl.ds(0, nr)])
            pltpu.sync_copy(dst_hbm.at[w, pl.ds(off, nr)], dstr.at[pl.ds(0, nr)])

    _stage(0, C0)
    pltpu.async_copy(rows0, acc_s.at[dst_c0.at[0]], ssc0, add=True)
    pltpu.async_copy(rows1, acc_s.at[dst_c0.at[0]], ssc1, add=True)
    pltpu.async_copy(rows2, acc_s.at[dst_c0.at[0]], ssc2, add=True)
    pltpu.async_copy(ex0, dcol_s.at[dst_c0.at[0]], sdc0, add=True)
    pltpu.async_copy(ex1, dcol_s.at[dst_c0.at[0]], sdc1, add=True)
    pltpu.async_copy(ex2, dcol_s.at[dst_c0.at[0]], sdc2, add=True)
    _issue(C0, 0, B0)
    _issue(C0, 1, B1)

    for ck in range(7):
        nr = 24 if ck < 6 else 18
        last = nr // 3 - 1
        C = C0 if ck % 2 == 0 else C1
        Cn = C1 if ck % 2 == 0 else C0
        has_next = ck < 6

        def _tri(p, _, C=C, Cn=Cn, ck=ck, last=last, has_next=has_next):
            a = 3 * p
            _process(C, a, B0)
            _issue(C, a + 2, B2)
            _process(C, a + 1, B1)

            @pl.when(p < last)
            def _i0():
                _issue(C, a + 3, B0)
            if has_next:
                @pl.when(p == last)
                def _i0n():
                    _stage(ck + 1, Cn)
                    _issue(Cn, 0, B0)

            _process(C, a + 2, B2)

            @pl.when(p < last)
            def _i1():
                _issue(C, a + 4, B1)
            if has_next:
                @pl.when(p == last)
                def _i1n():
                    _issue(Cn, 1, B1)
            return 0
        lax.fori_loop(0, nr // 3, _tri, 0)

    pltpu.make_async_copy(rows0, acc_s.at[dst_c0.at[0]], ssc0).wait()
    pltpu.make_async_copy(rows1, acc_s.at[dst_c0.at[0]], ssc1).wait()
    pltpu.make_async_copy(rows2, acc_s.at[dst_c0.at[0]], ssc2).wait()
    pltpu.make_async_copy(ex0, dcol_s.at[dst_c0.at[0]], sdc0).wait()
    pltpu.make_async_copy(ex1, dcol_s.at[dst_c0.at[0]], sdc1).wait()
    pltpu.make_async_copy(ex2, dcol_s.at[dst_c0.at[0]], sdc2).wait()
    plsc.subcore_barrier()

    pltpu.sync_copy(acc_s.at[pl.ds(s * SL, SL)],
                    outn_hbm.at[pl.ds(c * NPAD + s * SL, SL)])
    pltpu.sync_copy(dcol_s.at[pl.ds(s * SL, SL)],
                    outd_hbm.at[pl.ds(c * NPAD + s * SL, SL)])


@functools.partial(jax.jit, static_argnames=())
def _sc_edge(h, src2, dst2, a_s, a_d, ub):
    mesh = plsc.VectorSubcoreMesh(core_axis_name="c", subcore_axis_name="s")
    k = pl.kernel(
        _sc_body,
        out_type=(
            jax.ShapeDtypeStruct((2 * NPAD, EMB), jnp.float32),
            jax.ShapeDtypeStruct((2 * NPAD,), jnp.float32),
        ),
        mesh=mesh,
        compiler_params=pltpu.CompilerParams(needs_layout_passes=False),
        scratch_types=(
            [pltpu.VMEM((24, CR), jnp.int32)] * 4 +
            [pltpu.VMEM((CR, EMB), jnp.float32)] * 3 +
            [pltpu.VMEM((CR,), jnp.float32)] * 9 +
            [pltpu.VMEM((SL,), jnp.float32),
             pltpu.VMEM((16,), jnp.float32)] +
            [pltpu.SemaphoreType.DMA] * 15 +
            [pltpu.VMEM_SHARED((NPAD, EMB), jnp.float32),
             pltpu.VMEM_SHARED((NPAD,), jnp.float32)]
        ),
    )
    return k(h, src2, dst2, a_s, a_d, ub)


def kernel(input_node, inputad, res, resmask, W1, att_src1, att_dst1, b1,
           W2, att_src2, att_dst2, b2, lin_w, lin_b):
    npad = EP - (inputad.shape[1] + N)
    loop = jnp.arange(N, dtype=jnp.int32)
    padi = jnp.arange(npad, dtype=jnp.int32)
    src = jnp.concatenate([inputad[0], loop, padi % 128])
    dst = jnp.concatenate([inputad[1], loop, N + padi % (NPAD - N)])
    src2 = src.reshape(32, TBLK, CR)
    dst2 = dst.reshape(32, TBLK, CR)

    h1, as1, ad1, ub1 = _tc_pre(input_node, W1, att_src1, att_dst1)
    n1a, d1a = _sc_edge(h1, src2, dst2, as1, ad1, ub1)
    h2, as2, ad2, ub2 = _tc_mid(n1a, d1a, b1, W2, att_src2, att_dst2)
    n2a, d2a = _sc_edge(h2, src2, dst2, as2, ad2, ub2)
    x2, sm, loss = _tc_fin(n2a, d2a, b2, lin_w, lin_b, res, resmask)
    return (loss, sm, x2, resmask, x2)


# speedup vs baseline: 51.8505x; 1.0536x over previous
import functools

import jax
import jax.numpy as jnp
from jax import lax
from jax.experimental import pallas as pl
from jax.experimental.pallas import tpu as pltpu
from jax.experimental.pallas import tpu_sc as plsc

N = 10000
EMB = 128
NEG_SLOPE = 0.2
NPAD = 10240
EP = 335872
CR = 64
TBLK = EP // (32 * CR)
NCHUNK = 7
CLAST = 20
SL = NPAD // 16


def _proj(h, asv_ref, adv_ref):
    a_s = jnp.dot(h, asv_ref[:], preferred_element_type=jnp.float32)
    a_d = jnp.dot(h, adv_ref[:], preferred_element_type=jnp.float32)
    pad = jnp.zeros((NPAD - N, 1), jnp.float32)
    ub = jnp.max(a_s) + jnp.max(a_d)
    ub = jnp.where(ub < 0, NEG_SLOPE * ub, ub)
    return (jnp.concatenate([a_s, pad], axis=0),
            jnp.concatenate([a_d, pad], axis=0),
            jnp.full((1, 16), ub, jnp.float32))


def _pre_body(x_ref, w_ref, asv_ref, adv_ref, h_ref, as_ref, ad_ref, ub_ref):
    h = jnp.dot(x_ref[:], w_ref[:], preferred_element_type=jnp.float32)
    h_ref[:] = h
    as_ref[:], ad_ref[:], ub_ref[:] = _proj(h, asv_ref, adv_ref)


def _tc_pre(x, W, att_src, att_dst):
    h, a_s, a_d, ub = pl.pallas_call(
        _pre_body,
        out_shape=(
            jax.ShapeDtypeStruct((N, EMB), jnp.float32),
            jax.ShapeDtypeStruct((NPAD, 1), jnp.float32),
            jax.ShapeDtypeStruct((NPAD, 1), jnp.float32),
            jax.ShapeDtypeStruct((1, 16), jnp.float32),
        ),
    )(x, W, att_src.reshape(EMB, 1), att_dst.reshape(EMB, 1))
    return h, a_s[:, 0], a_d[:, 0], ub.reshape(16)


def _mid_body(n_ref, d_ref, b_ref, w_ref, asv_ref, adv_ref,
              h_ref, as_ref, ad_ref, ub_ref):
    inv = 1.0 / (d_ref[0:N] + d_ref[NPAD:NPAD + N])
    x = jax.nn.relu((n_ref[0:N] + n_ref[NPAD:NPAD + N]) * inv + b_ref[:])
    h = jnp.dot(x, w_ref[:], preferred_element_type=jnp.float32)
    h_ref[:] = h
    as_ref[:], ad_ref[:], ub_ref[:] = _proj(h, asv_ref, adv_ref)


def _tc_mid(n, d, b, W, att_src, att_dst):
    h, a_s, a_d, ub = pl.pallas_call(
        _mid_body,
        out_shape=(
            jax.ShapeDtypeStruct((N, EMB), jnp.float32),
            jax.ShapeDtypeStruct((NPAD, 1), jnp.float32),
            jax.ShapeDtypeStruct((NPAD, 1), jnp.float32),
            jax.ShapeDtypeStruct((1, 16), jnp.float32),
        ),
    )(n, d.reshape(2 * NPAD, 1), b.reshape(1, EMB), W,
      att_src.reshape(EMB, 1), att_dst.reshape(EMB, 1))
    return h, a_s[:, 0], a_d[:, 0], ub.reshape(16)


def _fin_body(n_ref, d_ref, b_ref, lw_ref, lb_ref,
              res_ref, rm_ref, x2_ref, sm_ref, loss_ref):
    inv = 1.0 / (d_ref[0:N] + d_ref[NPAD:NPAD + N])
    x2 = jax.nn.relu((n_ref[0:N] + n_ref[NPAD:NPAD + N]) * inv + b_ref[:])
    x2_ref[:] = x2
    li = jnp.dot(x2, lw_ref[:], preferred_element_type=jnp.float32) + lb_ref[0, 0]
    li = jnp.where(rm_ref[:] == 0, -1e9, li)
    m = jnp.max(li)
    ex = jnp.exp(li - m)
    sm = ex / jnp.sum(ex)
    sm_ref[:] = sm
    loss = -jnp.sum(jnp.log(jnp.clip(sm, 1e-10, 1.0)) * res_ref[:])
    loss_ref[:] = jnp.full((1, 1), loss, jnp.float32)


def _tc_fin(n, d, b, lin_w, lin_b, res, resmask):
    x2, sm, loss = pl.pallas_call(
        _fin_body,
        out_shape=(
            jax.ShapeDtypeStruct((N, EMB), jnp.float32),
            jax.ShapeDtypeStruct((N, 1), jnp.float32),
            jax.ShapeDtypeStruct((1, 1), jnp.float32),
        ),
    )(n, d.reshape(2 * NPAD, 1), b.reshape(1, EMB),
      lin_w, lin_b.reshape(1, 1), res.reshape(N, 1), resmask.reshape(N, 1))
    return x2, sm[:, 0], loss.reshape(())


def _sc_body(h_hbm, src_hbm, dst_hbm, as_hbm, ad_hbm, ub_hbm,
             outn_hbm, outd_hbm,
             src_c0, dst_c0, src_c1, dst_c1,
             rows0, rows1, rows2, rows3,
             asg0, adg0, ex0, asg1, adg1, ex1, asg2, adg2, ex2,
             asg3, adg3, ex3,
             zer_v, ub_v,
             sas0, sad0, srw0, ssc0, sdc0,
             sas1, sad1, srw1, ssc1, sdc1,
             sas2, sad2, srw2, ssc2, sdc2,
             sas3, sad3, srw3, ssc3, sdc3,
             acc_s, dcol_s):
    c = lax.axis_index("c")
    s = lax.axis_index("s")
    w = c * 16 + s

    pltpu.sync_copy(ub_hbm, ub_v)

    def _zr(i, _):
        for q in range(8):
            rows0[i, pl.ds(q * 16, 16)] = jnp.zeros((16,), jnp.float32)
            rows1[i, pl.ds(q * 16, 16)] = jnp.zeros((16,), jnp.float32)
            rows2[i, pl.ds(q * 16, 16)] = jnp.zeros((16,), jnp.float32)
            rows3[i, pl.ds(q * 16, 16)] = jnp.zeros((16,), jnp.float32)
        return 0
    lax.fori_loop(0, CR, _zr, 0)
    for q in range(CR // 16):
        ex0[pl.ds(q * 16, 16)] = jnp.zeros((16,), jnp.float32)
        ex1[pl.ds(q * 16, 16)] = jnp.zeros((16,), jnp.float32)
        ex2[pl.ds(q * 16, 16)] = jnp.zeros((16,), jnp.float32)
        ex3[pl.ds(q * 16, 16)] = jnp.zeros((16,), jnp.float32)

    def _zd(i, _):
        zer_v[pl.ds(i * 16, 16)] = jnp.zeros((16,), jnp.float32)
        return 0
    lax.fori_loop(0, SL // 16, _zd, 0)

    for t in range(SL // CR):
        pltpu.sync_copy(rows0, acc_s.at[pl.ds(s * SL + t * CR, CR)])
    if SL % CR:
        pltpu.sync_copy(rows0.at[pl.ds(0, SL % CR)],
                        acc_s.at[pl.ds(s * SL + SL - SL % CR, SL % CR)])
    pltpu.sync_copy(zer_v, dcol_s.at[pl.ds(s * SL, SL)])
    plsc.subcore_barrier()

    ub16 = ub_v[:]

    B0 = (rows0, asg0, adg0, ex0, sas0, sad0, srw0, ssc0, sdc0)
    B1 = (rows1, asg1, adg1, ex1, sas1, sad1, srw1, ssc1, sdc1)
    B2 = (rows2, asg2, adg2, ex2, sas2, sad2, srw2, ssc2, sdc2)
    B3 = (rows3, asg3, adg3, ex3, sas3, sad3, srw3, ssc3, sdc3)
    BUFS = (B0, B1, B2, B3)
    C0 = (src_c0, dst_c0)
    C1 = (src_c1, dst_c1)

    def _issue(C, rr, B):
        rows, asg, adg, ex, sas, sad, srw, ssc, sdc = B
        srcr, dstr = C
        pltpu.make_async_copy(rows, acc_s.at[dstr.at[rr]], ssc).wait()
        pltpu.make_async_copy(ex, dcol_s.at[dstr.at[rr]], sdc).wait()
        pltpu.async_copy(as_hbm.at[srcr.at[rr]], asg, sas)
        pltpu.async_copy(ad_hbm.at[dstr.at[rr]], adg, sad)
        pltpu.async_copy(h_hbm.at[srcr.at[rr]], rows, srw)

    def _process(C, rr, B):
        rows, asg, adg, ex, sas, sad, srw, ssc, sdc = B
        srcr, dstr = C
        pltpu.make_async_copy(as_hbm.at[srcr.at[rr]], asg, sas).wait()
        pltpu.make_async_copy(ad_hbm.at[dstr.at[rr]], adg, sad).wait()
        for q in range(CR // 16):
            e = asg[pl.ds(q * 16, 16)] + adg[pl.ds(q * 16, 16)]
            e = jnp.where(e < 0, NEG_SLOPE * e, e) - ub16
            ex[pl.ds(q * 16, 16)] = jnp.exp(e)
        pltpu.make_async_copy(h_hbm.at[srcr.at[rr]], rows, srw).wait()

        def _scale(r4, _):
            for u in range(4):
                r = r4 * 4 + u
                ev = plsc.load_gather(ex, [jnp.full((16,), r, jnp.int32)])
                for q in range(8):
                    rows[r, pl.ds(q * 16, 16)] = rows[r, pl.ds(q * 16, 16)] * ev
            return 0
        lax.fori_loop(0, CR // 4, _scale, 0)

        pltpu.async_copy(rows, acc_s.at[dstr.at[rr]], ssc, add=True)
        pltpu.async_copy(ex, dcol_s.at[dstr.at[rr]], sdc, add=True)

    def _stage(ck, C):
        srcr, dstr = C
        nr = 24 if ck < NCHUNK - 1 else CLAST
        off = ck * 24
        if nr == 24:
            pltpu.sync_copy(src_hbm.at[w, pl.ds(off, nr)], srcr)
            pltpu.sync_copy(dst_hbm.at[w, pl.ds(off, nr)], dstr)
        else:
            pltpu.sync_copy(src_hbm.at[w, pl.ds(off, nr)], srcr.at[pl.ds(0, nr)])
            pltpu.sync_copy(dst_hbm.at[w, pl.ds(off, nr)], dstr.at[pl.ds(0, nr)])

    _stage(0, C0)
    for B in BUFS:
        pltpu.async_copy(B[0], acc_s.at[dst_c0.at[0]], B[7], add=True)
        pltpu.async_copy(B[3], dcol_s.at[dst_c0.at[0]], B[8], add=True)
    _issue(C0, 0, B0)
    _issue(C0, 1, B1)
    _issue(C0, 2, B2)

    for ck in range(NCHUNK):
        nr = 24 if ck < NCHUNK - 1 else CLAST
        last = nr // 4 - 1
        C = C0 if ck % 2 == 0 else C1
        Cn = C1 if ck % 2 == 0 else C0
        has_next = ck < NCHUNK - 1

        def _quad(p, _, C=C, Cn=Cn, ck=ck, last=last, has_next=has_next):
            a = 4 * p
            _process(C, a, B0)
            _issue(C, a + 3, B3)
            _process(C, a + 1, B1)

            @pl.when(p < last)
            def _i0():
                _issue(C, a + 4, B0)
            if has_next:
                @pl.when(p == last)
                def _i0n():
                    _stage(ck + 1, Cn)
                    _issue(Cn, 0, B0)

            _process(C, a + 2, B2)

            @pl.when(p < last)
            def _i1():
                _issue(C, a + 5, B1)
            if has_next:
                @pl.when(p == last)
                def _i1n():
                    _issue(Cn, 1, B1)

            _process(C, a + 3, B3)

            @pl.when(p < last)
            def _i2():
                _issue(C, a + 6, B2)
            if has_next:
                @pl.when(p == last)
                def _i2n():
                    _issue(Cn, 2, B2)
            return 0
        lax.fori_loop(0, nr // 4, _quad, 0)

    for B in BUFS:
        pltpu.make_async_copy(B[0], acc_s.at[dst_c0.at[0]], B[7]).wait()
        pltpu.make_async_copy(B[3], dcol_s.at[dst_c0.at[0]], B[8]).wait()
    plsc.subcore_barrier()

    pltpu.sync_copy(acc_s.at[pl.ds(s * SL, SL)],
                    outn_hbm.at[pl.ds(c * NPAD + s * SL, SL)])
    pltpu.sync_copy(dcol_s.at[pl.ds(s * SL, SL)],
                    outd_hbm.at[pl.ds(c * NPAD + s * SL, SL)])


@functools.partial(jax.jit, static_argnames=())
def _sc_edge(h, src2, dst2, a_s, a_d, ub):
    mesh = plsc.VectorSubcoreMesh(core_axis_name="c", subcore_axis_name="s")
    k = pl.kernel(
        _sc_body,
        out_type=(
            jax.ShapeDtypeStruct((2 * NPAD, EMB), jnp.float32),
            jax.ShapeDtypeStruct((2 * NPAD,), jnp.float32),
        ),
        mesh=mesh,
        compiler_params=pltpu.CompilerParams(needs_layout_passes=False),
        scratch_types=(
            [pltpu.VMEM((24, CR), jnp.int32)] * 4 +
            [pltpu.VMEM((CR, EMB), jnp.float32)] * 4 +
            [pltpu.VMEM((CR,), jnp.float32)] * 12 +
            [pltpu.VMEM((SL,), jnp.float32),
             pltpu.VMEM((16,), jnp.float32)] +
            [pltpu.SemaphoreType.DMA] * 20 +
            [pltpu.VMEM_SHARED((NPAD, EMB), jnp.float32),
             pltpu.VMEM_SHARED((NPAD,), jnp.float32)]
        ),
    )
    return k(h, src2, dst2, a_s, a_d, ub)


def kernel(input_node, inputad, res, resmask, W1, att_src1, att_dst1, b1,
           W2, att_src2, att_dst2, b2, lin_w, lin_b):
    npad = EP - (inputad.shape[1] + N)
    loop = jnp.arange(N, dtype=jnp.int32)
    padi = jnp.arange(npad, dtype=jnp.int32)
    src = jnp.concatenate([inputad[0], loop, padi % 128])
    dst = jnp.concatenate([inputad[1], loop, N + padi % (NPAD - N)])
    src2 = src.reshape(32, TBLK, CR)
    dst2 = dst.reshape(32, TBLK, CR)
    assert TBLK == (NCHUNK - 1) * 24 + CLAST

    h1, as1, ad1, ub1 = _tc_pre(input_node, W1, att_src1, att_dst1)
    n1a, d1a = _sc_edge(h1, src2, dst2, as1, ad1, ub1)
    h2, as2, ad2, ub2 = _tc_mid(n1a, d1a, b1, W2, att_src2, att_dst2)
    n2a, d2a = _sc_edge(h2, src2, dst2, as2, ad2, ub2)
    x2, sm, loss = _tc_fin(n2a, d2a, b2, lin_w, lin_b, res, resmask)
    return (loss, sm, x2, resmask, x2)


# speedup vs baseline: 52.0772x; 1.0044x over previous
import functools

import jax
import jax.numpy as jnp
from jax import lax
from jax.experimental import pallas as pl
from jax.experimental.pallas import tpu as pltpu
from jax.experimental.pallas import tpu_sc as plsc

N = 10000
EMB = 128
NEG_SLOPE = 0.2
NPAD = 10240
EP = 335872
CR = 64
TBLK = EP // (32 * CR)
NCHUNK = 7
CLAST = 20
SL = NPAD // 16


def _proj(h, asv_ref, adv_ref):
    a_s = jnp.dot(h, asv_ref[:], preferred_element_type=jnp.float32)
    a_d = jnp.dot(h, adv_ref[:], preferred_element_type=jnp.float32)
    pad = jnp.zeros((NPAD - N, 1), jnp.float32)
    ub = jnp.max(a_s) + jnp.max(a_d)
    ub = jnp.where(ub < 0, NEG_SLOPE * ub, ub)
    return (jnp.concatenate([a_s, pad], axis=0),
            jnp.concatenate([a_d, pad], axis=0),
            jnp.full((1, 16), ub, jnp.float32))


def _pre_body(x_ref, w_ref, asv_ref, adv_ref, h_ref, as_ref, ad_ref, ub_ref):
    h = jnp.dot(x_ref[:], w_ref[:], preferred_element_type=jnp.float32)
    h_ref[:] = h
    as_ref[:], ad_ref[:], ub_ref[:] = _proj(h, asv_ref, adv_ref)


def _tc_pre(x, W, att_src, att_dst):
    h, a_s, a_d, ub = pl.pallas_call(
        _pre_body,
        out_shape=(
            jax.ShapeDtypeStruct((N, EMB), jnp.float32),
            jax.ShapeDtypeStruct((NPAD, 1), jnp.float32),
            jax.ShapeDtypeStruct((NPAD, 1), jnp.float32),
            jax.ShapeDtypeStruct((1, 16), jnp.float32),
        ),
    )(x, W, att_src.reshape(EMB, 1), att_dst.reshape(EMB, 1))
    return h, a_s[:, 0], a_d[:, 0], ub.reshape(16)


def _mid_body(n_ref, d_ref, b_ref, w_ref, asv_ref, adv_ref,
              h_ref, as_ref, ad_ref, ub_ref):
    inv = 1.0 / (d_ref[0:N] + d_ref[NPAD:NPAD + N])
    x = jax.nn.relu((n_ref[0:N] + n_ref[NPAD:NPAD + N]) * inv + b_ref[:])
    h = jnp.dot(x, w_ref[:], preferred_element_type=jnp.float32)
    h_ref[:] = h
    as_ref[:], ad_ref[:], ub_ref[:] = _proj(h, asv_ref, adv_ref)


def _tc_mid(n, d, b, W, att_src, att_dst):
    h, a_s, a_d, ub = pl.pallas_call(
        _mid_body,
        out_shape=(
            jax.ShapeDtypeStruct((N, EMB), jnp.float32),
            jax.ShapeDtypeStruct((NPAD, 1), jnp.float32),
            jax.ShapeDtypeStruct((NPAD, 1), jnp.float32),
            jax.ShapeDtypeStruct((1, 16), jnp.float32),
        ),
    )(n, d.reshape(2 * NPAD, 1), b.reshape(1, EMB), W,
      att_src.reshape(EMB, 1), att_dst.reshape(EMB, 1))
    return h, a_s[:, 0], a_d[:, 0], ub.reshape(16)


def _fin_body(n_ref, d_ref, b_ref, lw_ref, lb_ref,
              res_ref, rm_ref, x2_ref, sm_ref, loss_ref):
    inv = 1.0 / (d_ref[0:N] + d_ref[NPAD:NPAD + N])
    x2 = jax.nn.relu((n_ref[0:N] + n_ref[NPAD:NPAD + N]) * inv + b_ref[:])
    x2_ref[:] = x2
    li = jnp.dot(x2, lw_ref[:], preferred_element_type=jnp.float32) + lb_ref[0, 0]
    li = jnp.where(rm_ref[:] == 0, -1e9, li)
    m = jnp.max(li)
    ex = jnp.exp(li - m)
    sm = ex / jnp.sum(ex)
    sm_ref[:] = sm
    loss = -jnp.sum(jnp.log(jnp.clip(sm, 1e-10, 1.0)) * res_ref[:])
    loss_ref[:] = jnp.full((1, 1), loss, jnp.float32)


def _tc_fin(n, d, b, lin_w, lin_b, res, resmask):
    x2, sm, loss = pl.pallas_call(
        _fin_body,
        out_shape=(
            jax.ShapeDtypeStruct((N, EMB), jnp.float32),
            jax.ShapeDtypeStruct((N, 1), jnp.float32),
            jax.ShapeDtypeStruct((1, 1), jnp.float32),
        ),
    )(n, d.reshape(2 * NPAD, 1), b.reshape(1, EMB),
      lin_w, lin_b.reshape(1, 1), res.reshape(N, 1), resmask.reshape(N, 1))
    return x2, sm[:, 0], loss.reshape(())


def _sc_body(h_hbm, src_hbm, dst_hbm, as_hbm, ad_hbm, ub_hbm,
             outn_hbm, outd_hbm,
             src_c0, dst_c0, src_c1, dst_c1,
             rows0, rows1, rows2, rows3,
             asg0, adg0, ex0, asg1, adg1, ex1, asg2, adg2, ex2,
             asg3, adg3, ex3,
             zer_v, ub_v,
             sas0, sad0, srw0, ssc0, sdc0,
             sas1, sad1, srw1, ssc1, sdc1,
             sas2, sad2, srw2, ssc2, sdc2,
             sas3, sad3, srw3, ssc3, sdc3,
             acc_s, dcol_s):
    c = lax.axis_index("c")
    s = lax.axis_index("s")
    w = c * 16 + s

    pltpu.sync_copy(ub_hbm, ub_v)

    def _zr(i, _):
        for q in range(8):
            rows0[i, pl.ds(q * 16, 16)] = jnp.zeros((16,), jnp.float32)
            rows1[i, pl.ds(q * 16, 16)] = jnp.zeros((16,), jnp.float32)
            rows2[i, pl.ds(q * 16, 16)] = jnp.zeros((16,), jnp.float32)
            rows3[i, pl.ds(q * 16, 16)] = jnp.zeros((16,), jnp.float32)
        return 0
    lax.fori_loop(0, CR, _zr, 0)
    for q in range(CR // 16):
        ex0[pl.ds(q * 16, 16)] = jnp.zeros((16,), jnp.float32)
        ex1[pl.ds(q * 16, 16)] = jnp.zeros((16,), jnp.float32)
        ex2[pl.ds(q * 16, 16)] = jnp.zeros((16,), jnp.float32)
        ex3[pl.ds(q * 16, 16)] = jnp.zeros((16,), jnp.float32)

    def _zd(i, _):
        zer_v[pl.ds(i * 16, 16)] = jnp.zeros((16,), jnp.float32)
        return 0
    lax.fori_loop(0, SL // 16, _zd, 0)

    zcps = [pltpu.async_copy(rows0, acc_s.at[pl.ds(s * SL + t * CR, CR)], srw1)
            for t in range(SL // CR)]
    if SL % CR:
        zcps.append(pltpu.async_copy(
            rows0.at[pl.ds(0, SL % CR)],
            acc_s.at[pl.ds(s * SL + SL - SL % CR, SL % CR)], srw1))
    zcps.append(pltpu.async_copy(zer_v, dcol_s.at[pl.ds(s * SL, SL)], srw2))
    for cp in zcps:
        cp.wait()
    plsc.subcore_barrier()

    ub16 = ub_v[:]

    B0 = (rows0, asg0, adg0, ex0, sas0, sad0, srw0, ssc0, sdc0)
    B1 = (rows1, asg1, adg1, ex1, sas1, sad1, srw1, ssc1, sdc1)
    B2 = (rows2, asg2, adg2, ex2, sas2, sad2, srw2, ssc2, sdc2)
    B3 = (rows3, asg3, adg3, ex3, sas3, sad3, srw3, ssc3, sdc3)
    BUFS = (B0, B1, B2, B3)
    C0 = (src_c0, dst_c0)
    C1 = (src_c1, dst_c1)

    def _issue(C, rr, B):
        rows, asg, adg, ex, sas, sad, srw, ssc, sdc = B
        srcr, dstr = C
        pltpu.make_async_copy(rows, acc_s.at[dstr.at[rr]], ssc).wait()
        pltpu.make_async_copy(ex, dcol_s.at[dstr.at[rr]], sdc).wait()
        pltpu.async_copy(as_hbm.at[srcr.at[rr]], asg, sas)
        pltpu.async_copy(ad_hbm.at[dstr.at[rr]], adg, sad)
        pltpu.async_copy(h_hbm.at[srcr.at[rr]], rows, srw)

    def _process(C, rr, B):
        rows, asg, adg, ex, sas, sad, srw, ssc, sdc = B
        srcr, dstr = C
        pltpu.make_async_copy(as_hbm.at[srcr.at[rr]], asg, sas).wait()
        pltpu.make_async_copy(ad_hbm.at[dstr.at[rr]], adg, sad).wait()
        for q in range(CR // 16):
            e = asg[pl.ds(q * 16, 16)] + adg[pl.ds(q * 16, 16)]
            e = jnp.where(e < 0, NEG_SLOPE * e, e) - ub16
            ex[pl.ds(q * 16, 16)] = jnp.exp(e)
        pltpu.make_async_copy(h_hbm.at[srcr.at[rr]], rows, srw).wait()

        def _scale(r4, _):
            for u in range(4):
                r = r4 * 4 + u
                ev = plsc.load_gather(ex, [jnp.full((16,), r, jnp.int32)])
                for q in range(8):
                    rows[r, pl.ds(q * 16, 16)] = rows[r, pl.ds(q * 16, 16)] * ev
            return 0
        lax.fori_loop(0, CR // 4, _scale, 0)

        pltpu.async_copy(rows, acc_s.at[dstr.at[rr]], ssc, add=True)
        pltpu.async_copy(ex, dcol_s.at[dstr.at[rr]], sdc, add=True)

    def _stage(ck, C):
        srcr, dstr = C
        nr = 24 if ck < NCHUNK - 1 else CLAST
        off = ck * 24
        if nr == 24:
            pltpu.sync_copy(src_hbm.at[w, pl.ds(off, nr)], srcr)
            pltpu.sync_copy(dst_hbm.at[w, pl.ds(off, nr)], dstr)
        else:
            pltpu.sync_copy(src_hbm.at[w, pl.ds(off, nr)], srcr.at[pl.ds(0, nr)])
            pltpu.sync_copy(dst_hbm.at[w, pl.ds(off, nr)], dstr.at[pl.ds(0, nr)])

    _stage(0, C0)
    for B in BUFS:
        pltpu.async_copy(B[0], acc_s.at[dst_c0.at[0]], B[7], add=True)
        pltpu.async_copy(B[3], dcol_s.at[dst_c0.at[0]], B[8], add=True)
    _issue(C0, 0, B0)
    _issue(C0, 1, B1)
    _issue(C0, 2, B2)

    for ck in range(NCHUNK):
        nr = 24 if ck < NCHUNK - 1 else CLAST
        last = nr // 4 - 1
        C = C0 if ck % 2 == 0 else C1
        Cn = C1 if ck % 2 == 0 else C0
        has_next = ck < NCHUNK - 1

        def _quad(p, _, C=C, Cn=Cn, ck=ck, last=last, has_next=has_next):
            a = 4 * p
            _process(C, a, B0)
            _issue(C, a + 3, B3)
            _process(C, a + 1, B1)

            @pl.when(p < last)
            def _i0():
                _issue(C, a + 4, B0)
            if has_next:
                @pl.when(p == last)
                def _i0n():
                    _stage(ck + 1, Cn)
                    _issue(Cn, 0, B0)

            _process(C, a + 2, B2)

            @pl.when(p < last)
            def _i1():
                _issue(C, a + 5, B1)
            if has_next:
                @pl.when(p == last)
                def _i1n():
                    _issue(Cn, 1, B1)

            _process(C, a + 3, B3)

            @pl.when(p < last)
            def _i2():
                _issue(C, a + 6, B2)
            if has_next:
                @pl.when(p == last)
                def _i2n():
                    _issue(Cn, 2, B2)
            return 0
        lax.fori_loop(0, nr // 4, _quad, 0)

    for B in BUFS:
        pltpu.make_async_copy(B[0], acc_s.at[dst_c0.at[0]], B[7]).wait()
        pltpu.make_async_copy(B[3], dcol_s.at[dst_c0.at[0]], B[8]).wait()
    plsc.subcore_barrier()

    pltpu.sync_copy(acc_s.at[pl.ds(s * SL, SL)],
                    outn_hbm.at[pl.ds(c * NPAD + s * SL, SL)])
    pltpu.sync_copy(dcol_s.at[pl.ds(s * SL, SL)],
                    outd_hbm.at[pl.ds(c * NPAD + s * SL, SL)])


@functools.partial(jax.jit, static_argnames=())
def _sc_edge(h, src2, dst2, a_s, a_d, ub):
    mesh = plsc.VectorSubcoreMesh(core_axis_name="c", subcore_axis_name="s")
    k = pl.kernel(
        _sc_body,
        out_type=(
            jax.ShapeDtypeStruct((2 * NPAD, EMB), jnp.float32),
            jax.ShapeDtypeStruct((2 * NPAD,), jnp.float32),
        ),
        mesh=mesh,
        compiler_params=pltpu.CompilerParams(needs_layout_passes=False),
        scratch_types=(
            [pltpu.VMEM((24, CR), jnp.int32)] * 4 +
            [pltpu.VMEM((CR, EMB), jnp.float32)] * 4 +
            [pltpu.VMEM((CR,), jnp.float32)] * 12 +
            [pltpu.VMEM((SL,), jnp.float32),
             pltpu.VMEM((16,), jnp.float32)] +
            [pltpu.SemaphoreType.DMA] * 20 +
            [pltpu.VMEM_SHARED((NPAD, EMB), jnp.float32),
             pltpu.VMEM_SHARED((NPAD,), jnp.float32)]
        ),
    )
    return k(h, src2, dst2, a_s, a_d, ub)


def kernel(input_node, inputad, res, resmask, W1, att_src1, att_dst1, b1,
           W2, att_src2, att_dst2, b2, lin_w, lin_b):
    npad = EP - (inputad.shape[1] + N)
    loop = jnp.arange(N, dtype=jnp.int32)
    padi = jnp.arange(npad, dtype=jnp.int32)
    src = jnp.concatenate([inputad[0], loop, padi % 128])
    dst = jnp.concatenate([inputad[1], loop, N + padi % (NPAD - N)])
    src2 = src.reshape(32, TBLK, CR)
    dst2 = dst.reshape(32, TBLK, CR)
    assert TBLK == (NCHUNK - 1) * 24 + CLAST

    h1, as1, ad1, ub1 = _tc_pre(input_node, W1, att_src1, att_dst1)
    n1a, d1a = _sc_edge(h1, src2, dst2, as1, ad1, ub1)
    h2, as2, ad2, ub2 = _tc_mid(n1a, d1a, b1, W2, att_src2, att_dst2)
    n2a, d2a = _sc_edge(h2, src2, dst2, as2, ad2, ub2)
    x2, sm, loss = _tc_fin(n2a, d2a, b2, lin_w, lin_b, res, resmask)
    return (loss, sm, x2, resmask, x2)
